# baseline (speedup 1.0000x reference)
import functools

import jax
import jax.numpy as jnp
from jax import lax
from jax.experimental import pallas as pl
from jax.experimental.pallas import tpu as pltpu
from jax.experimental.pallas import tpu_sc as plsc

_N = 4096
_EXT_E = 131072 + _N
_ROWS128 = _EXT_E // 128
_RPT_DEG = _ROWS128 // 32
_RPT_A = _ROWS128 // 16
_BLK_ROWS = 256
_BLK_WORDS = _BLK_ROWS * _N
_SHARE = _BLK_WORDS // 16


@functools.cache
def _deg_kernel():
    return functools.partial(
        pl.kernel,
        out_type=jax.ShapeDtypeStruct((2 * _N,), jnp.float32),
        mesh=plsc.VectorSubcoreMesh(core_axis_name="c", subcore_axis_name="s"),
        compiler_params=pltpu.CompilerParams(needs_layout_passes=False),
        scratch_types=[
            pltpu.VMEM((_RPT_DEG * 128,), jnp.int32),
            pltpu.VMEM((_RPT_DEG, 128), jnp.int32),
            pltpu.VMEM((128,), jnp.float32),
            pltpu.VMEM((_N,), jnp.float32),
            pltpu.VMEM_SHARED((_N,), jnp.float32),
        ],
    )(_deg_body)


def _deg_body(dst_hbm, out_hbm, dstf, dstb, ones_v, zv, acc_sh):
    c = lax.axis_index("c")
    s = lax.axis_index("s")
    shard = c * 16 + s
    nwords = _RPT_DEG * 128

    def _fill(i, carry):
        ones_v[pl.ds(i * 16, 16)] = jnp.full((16,), 1.0, jnp.float32)
        return carry

    lax.fori_loop(0, 8, _fill, 0)

    def _zero(i, carry):
        zv[pl.ds(i * 16, 16)] = jnp.zeros((16,), jnp.float32)
        return carry

    lax.fori_loop(0, _N // 16, _zero, 0)

    @pl.when(s == 0)
    def _():
        pltpu.sync_copy(zv, acc_sh)

    plsc.subcore_barrier()
    pltpu.sync_copy(dst_hbm.at[pl.ds(shard * nwords, nwords)], dstf)

    def _pack(j, carry):
        for t in range(8):
            dstb[j, pl.ds(t * 16, 16)] = dstf[pl.ds(j * 128 + t * 16, 16)]
        return carry

    lax.fori_loop(0, _RPT_DEG, _pack, 0)

    def _scat(j, carry):
        pltpu.sync_copy(ones_v, acc_sh.at[dstb.at[j]], add=True)
        return carry

    lax.fori_loop(0, _RPT_DEG, _scat, 0)
    plsc.subcore_barrier()

    @pl.when(s == 0)
    def _():
        pltpu.sync_copy(acc_sh, out_hbm.at[pl.ds(c * _N, _N)])


@functools.cache
def _abuild_kernel():
    return functools.partial(
        pl.kernel,
        out_type=jax.ShapeDtypeStruct((_N * _N,), jnp.float32),
        mesh=plsc.VectorSubcoreMesh(core_axis_name="c", subcore_axis_name="s"),
        compiler_params=pltpu.CompilerParams(needs_layout_passes=False),
        scratch_types=[
            pltpu.VMEM((_N,), jnp.float32),
            pltpu.VMEM((_RPT_A * 128,), jnp.int32),
            pltpu.VMEM((_RPT_A * 128,), jnp.int32),
            pltpu.VMEM((_RPT_A * 128,), jnp.int32),
            pltpu.VMEM((_RPT_A * 128,), jnp.float32),
            pltpu.VMEM((_RPT_A, 128), jnp.int32),
            pltpu.VMEM((_RPT_A * 128,), jnp.float32),
            pltpu.VMEM((8192,), jnp.float32),
            pltpu.VMEM_SHARED((_BLK_WORDS,), jnp.float32),
            pltpu.SemaphoreType.DMA,
        ],
    )(_abuild_body)


def _abuild_body(src_hbm, dst_hbm, dis_hbm, a_hbm,
                 dis_v, srcb, dstb, gab, nrmb, idxb, valb, zv, acc_sh, sem):
    c = lax.axis_index("c")
    s = lax.axis_index("s")
    pltpu.sync_copy(dis_hbm, dis_v)

    def _zfill(i, carry):
        zv[pl.ds(i * 16, 16)] = jnp.zeros((16,), jnp.float32)
        return carry

    lax.fori_loop(0, 8192 // 16, _zfill, 0)

    nwords = _RPT_A * 128
    pltpu.sync_copy(src_hbm.at[pl.ds(s * nwords, nwords)], srcb)
    pltpu.sync_copy(dst_hbm.at[pl.ds(s * nwords, nwords)], dstb)

    def _pre(v, carry):
        off = v * 16
        sv = srcb[pl.ds(off, 16)]
        dv = dstb[pl.ds(off, 16)]
        nrm = plsc.load_gather(dis_v, [sv]) * plsc.load_gather(dis_v, [dv])
        ga = (dv << 12) + sv
        nrmb[pl.ds(off, 16)] = nrm
        gab[pl.ds(off, 16)] = ga
        return carry

    lax.fori_loop(0, nwords // 16, _pre, 0)

    def _pass(p, carry):
        boff = (c * 8 + p) * _BLK_WORDS

        def _zero(q, c2):
            pltpu.sync_copy(zv, acc_sh.at[pl.ds(s * _SHARE + q * 8192, 8192)])
            return c2

        lax.fori_loop(0, _SHARE // 8192, _zero, 0)
        plsc.subcore_barrier()

        def _cmp(j, c2):
            for t in range(8):
                off = j * 128 + t * 16
                ga = gab[pl.ds(off, 16)]
                idx = ga - boff
                inb = (idx >= 0) & (idx < _BLK_WORDS)
                idxs = jnp.where(inb, idx, ga & 1023)
                val = jnp.where(inb, nrmb[pl.ds(off, 16)],
                                jnp.zeros((16,), jnp.float32))
                idxb[j, pl.ds(t * 16, 16)] = idxs
                valb[pl.ds(off, 16)] = val
            return c2

        lax.fori_loop(0, _RPT_A, _cmp, 0)

        handles = []
        for j in range(_RPT_A):
            handles.append(pltpu.async_copy(
                valb.at[pl.ds(j * 128, 128)],
                acc_sh.at[idxb.at[j]], sem, add=True))
            if j >= 4:
                handles[j - 4].wait()
        for h in handles[-4:]:
            h.wait()
        plsc.subcore_barrier()
        pltpu.sync_copy(acc_sh.at[pl.ds(s * _SHARE, _SHARE)],
                        a_hbm.at[pl.ds(boff + s * _SHARE, _SHARE)])
        plsc.subcore_barrier()
        return carry

    lax.fori_loop(0, 8, _pass, 0)


def _dis_call(deg_parts):
    def body(deg_ref, o_ref):
        o_ref[...] = lax.rsqrt(jnp.sum(deg_ref[...], axis=0, keepdims=True))

    return pl.pallas_call(
        body, out_shape=jax.ShapeDtypeStruct((1, _N), jnp.float32)
    )(deg_parts)


def _matmul(a, b, *, bias=None, add_a=None, add_b=None, trans_b=False,
            relu=False, bm=1024, bn=512, bk=512, a_flat=False):
    if a_flat:
        K = b.shape[1] if trans_b else b.shape[0]
        M = a.size // K
    else:
        M, K = a.shape
    Nn = b.shape[0] if trans_b else b.shape[1]
    bm, bn, bk = min(bm, M), min(bn, Nn), min(bk, K)
    assert not a_flat or bk == K
    nk = K // bk
    grid = (M // bm, Nn // bn, nk)
    dims = (((1,), (1,)), ((), ())) if trans_b else (((1,), (0,)), ((), ()))

    if a_flat:
        a_spec = pl.BlockSpec((bm * K // 128, 128), lambda i, j, k: (i, 0))
    else:
        a_spec = pl.BlockSpec((bm, bk), lambda i, j, k: (i, k))
    b_spec = (pl.BlockSpec((bn, bk), lambda i, j, k: (j, k)) if trans_b
              else pl.BlockSpec((bk, bn), lambda i, j, k: (k, j)))
    in_specs = [a_spec, b_spec]
    args = [a, b]
    if add_a is not None:
        in_specs.append(a_spec)
        args.append(add_a)
    if add_b is not None:
        in_specs.append(b_spec)
        args.append(add_b)
    if bias is not None:
        in_specs.append(pl.BlockSpec((1, bn), lambda i, j, k: (0, j)))
        args.append(bias.reshape(1, Nn))

    has_add_a, has_add_b, has_bias = (add_a is not None, add_b is not None,
                                      bias is not None)

    def body(*refs):
        if nk > 1:
            *ins, o_ref, acc_ref = refs
        else:
            *ins, o_ref = refs
            acc_ref = None
        it = iter(ins)
        a_ref, b_ref = next(it), next(it)
        a2_ref = next(it) if has_add_a else None
        b2_ref = next(it) if has_add_b else None
        bias_ref = next(it) if has_bias else None

        av = a_ref.reshape(bm, bk)[...] if a_flat else a_ref[...]
        if has_add_a:
            av = av + a2_ref[...]
        bv = b_ref[...]
        if has_add_b:
            bv = bv + b2_ref[...]
        part = lax.dot_general(av, bv, dims,
                               preferred_element_type=jnp.float32)

        def _epilogue(r):
            if has_bias:
                r = r + bias_ref[...]
            if relu:
                r = jnp.maximum(r, 0.0)
            o_ref[...] = r

        if nk > 1:
            k = pl.program_id(2)

            @pl.when(k == 0)
            def _():
                acc_ref[...] = part

            @pl.when(k > 0)
            def _():
                acc_ref[...] += part

            @pl.when(k == nk - 1)
            def _():
                _epilogue(acc_ref[...])
        else:
            _epilogue(part)

    scratch = [pltpu.VMEM((bm, bn), jnp.float32)] if nk > 1 else []
    return pl.pallas_call(
        body, grid=grid,
        in_specs=in_specs,
        out_specs=pl.BlockSpec((bm, bn), lambda i, j, k: (i, j)),
        out_shape=jax.ShapeDtypeStruct((M, Nn), jnp.float32),
        scratch_shapes=scratch,
        compiler_params=pltpu.CompilerParams(
            dimension_semantics=("parallel", "parallel", "arbitrary"),
            vmem_limit_bytes=100 * 1024 * 1024),
    )(*args)


def _mlp_chain(B, ws, bs, bm=512):
    grid = (_N // bm,)
    in_specs = [pl.BlockSpec((bm, _N), lambda i: (i, 0))]
    args = [B]
    for w, b in zip(ws, bs):
        in_specs.append(pl.BlockSpec(w.shape, lambda i: (0, 0)))
        args.append(w)
        in_specs.append(pl.BlockSpec((1, b.shape[0]), lambda i: (0, 0)))
        args.append(b.reshape(1, -1))

    out_shapes = [jax.ShapeDtypeStruct((_N, ws[0].shape[1]), jnp.float32),
                  jax.ShapeDtypeStruct((_N, ws[1].shape[1]), jnp.float32),
                  jax.ShapeDtypeStruct((_N, ws[2].shape[1]), jnp.float32),
                  jax.ShapeDtypeStruct((_N, ws[5].shape[1]), jnp.float32)]
    out_specs = [pl.BlockSpec((bm, s.shape[1]), lambda i: (i, 0))
                 for s in out_shapes]

    def body(*refs):
        b_ref = refs[0]
        wrefs = refs[1:13]
        h1_ref, h2_ref, h3_ref, bhat_ref = refs[13:]
        h = b_ref[...]
        outs = []
        for li in range(6):
            h = jnp.maximum(
                jnp.dot(h, wrefs[2 * li][...],
                        preferred_element_type=jnp.float32)
                + wrefs[2 * li + 1][...], 0.0)
            outs.append(h)
        h1_ref[...] = outs[0]
        h2_ref[...] = outs[1]
        h3_ref[...] = outs[2]
        bhat_ref[...] = outs[5]

    return pl.pallas_call(
        body, grid=grid,
        in_specs=in_specs,
        out_specs=out_specs,
        out_shape=out_shapes,
        compiler_params=pltpu.CompilerParams(
            dimension_semantics=("parallel",),
            vmem_limit_bytes=100 * 1024 * 1024),
    )(*args)


def _a_stage(A, rhs, *, w1=None, bias=None, relu=False, add=None, w2=None,
             extra_w=None, bm=512):
    Kr = rhs.shape[1]
    M = _N
    n_mid = w1.shape[1] if w1 is not None else Kr
    n_out = w2.shape[1] if w2 is not None else n_mid
    grid = (M // bm,)

    in_specs = [pl.BlockSpec((bm * _N // 128, 128), lambda i: (i, 0)),
                pl.BlockSpec((_N, Kr), lambda i: (0, 0))]
    args = [A, rhs]
    if w1 is not None:
        in_specs.append(pl.BlockSpec(w1.shape, lambda i: (0, 0)))
        args.append(w1)
    if bias is not None:
        in_specs.append(pl.BlockSpec((1, n_mid), lambda i: (0, 0)))
        args.append(bias.reshape(1, n_mid))
    if add is not None:
        in_specs.append(pl.BlockSpec((bm, n_mid), lambda i: (i, 0)))
        args.append(add)
    if w2 is not None:
        in_specs.append(pl.BlockSpec(w2.shape, lambda i: (0, 0)))
        args.append(w2)
    if extra_w is not None:
        in_specs.append(pl.BlockSpec(extra_w.shape, lambda i: (0, 0)))
        args.append(extra_w)

    out_shapes = [jax.ShapeDtypeStruct((M, n_out), jnp.float32)]
    out_specs = [pl.BlockSpec((bm, n_out), lambda i: (i, 0))]
    if extra_w is not None:
        out_shapes.append(
            jax.ShapeDtypeStruct((M, extra_w.shape[1]), jnp.float32))
        out_specs.append(
            pl.BlockSpec((bm, extra_w.shape[1]), lambda i: (i, 0)))

    has = (w1 is not None, bias is not None, add is not None,
           w2 is not None, extra_w is not None)

    def body(*refs):
        nouts = 2 if extra_w is not None else 1
        ins, outs = refs[:-nouts], refs[-nouts:]
        it = iter(ins)
        a_ref, rhs_ref = next(it), next(it)
        w1_ref = next(it) if has[0] else None
        b_ref = next(it) if has[1] else None
        add_ref = next(it) if has[2] else None
        w2_ref = next(it) if has[3] else None
        ew_ref = next(it) if has[4] else None

        acc = lax.dot_general(a_ref.reshape(bm, _N)[...], rhs_ref[...],
                              (((1,), (0,)), ((), ())),
                              preferred_element_type=jnp.float32)
        if has[0]:
            acc = jnp.dot(acc, w1_ref[...],
                          preferred_element_type=jnp.float32)
        if has[1]:
            acc = acc + b_ref[...]
        if relu:
            acc = jnp.maximum(acc, 0.0)
        if has[4]:
            outs[1][...] = jnp.dot(acc, ew_ref[...],
                                   preferred_element_type=jnp.float32)
        if has[2]:
            acc = acc + add_ref[...]
        if has[3]:
            acc = jnp.dot(acc, w2_ref[...],
                          preferred_element_type=jnp.float32)
        outs[0][...] = acc

    res = pl.pallas_call(
        body, grid=grid,
        in_specs=in_specs,
        out_specs=out_specs if extra_w is not None else out_specs[0],
        out_shape=out_shapes if extra_w is not None else out_shapes[0],
        compiler_params=pltpu.CompilerParams(
            dimension_semantics=("parallel",),
            vmem_limit_bytes=100 * 1024 * 1024),
    )(*args)
    return res


def kernel(x, edge_index, B, W_enc0, b_enc0, W_enc1, b_enc1, W_enc2, b_enc2,
           W_dec0, b_dec0, W_dec1, b_dec1, W_dec2, b_dec2,
           W_gcn0, b_gcn0, W_gcn1, b_gcn1, W_gcn2, b_gcn2, W_gcn3, b_gcn3,
           W_attr, b_attr):
    loop = jnp.arange(_N, dtype=edge_index.dtype)
    src_e = jnp.concatenate([edge_index[0], loop])
    dst_e = jnp.concatenate([edge_index[1], loop])

    deg_parts = _deg_kernel()(dst_e).reshape(2, _N)
    dis = _dis_call(deg_parts).reshape(_N)
    A = _abuild_kernel()(src_e, dst_e, dis).reshape(_N * 32, 128)

    h1, h2, h3, B_hat = _mlp_chain(
        B,
        (W_enc0, W_enc1, W_enc2, W_dec0, W_dec1, W_dec2),
        (b_enc0, b_enc1, b_enc2, b_dec0, b_dec1, b_dec2))

    t1 = _a_stage(A, x, w1=W_gcn0, bias=b_gcn0, relu=True, add=h1)
    u2 = _a_stage(A, t1, w1=W_gcn1, bias=b_gcn1, relu=True, add=h2, w2=W_gcn2)
    u3 = _a_stage(A, u2, bias=b_gcn2, relu=True, add=h3, w2=W_gcn3)
    z, ua = _a_stage(A, u3, bias=b_gcn3, relu=True, extra_w=W_attr)
    X_hat = _a_stage(A, ua, bias=b_attr)

    A_hat = _matmul(z, z, trans_b=True, bm=1024, bn=1024)

    return (A_hat, B_hat, X_hat, h3, z)


# speedup vs baseline: 1.0200x; 1.0200x over previous
import functools

import jax
import jax.numpy as jnp
from jax import lax
from jax.experimental import pallas as pl
from jax.experimental.pallas import tpu as pltpu
from jax.experimental.pallas import tpu_sc as plsc

_N = 4096
_EXT_E = 131072 + _N
_ROWS128 = _EXT_E // 128
_RPT_DEG = _ROWS128 // 32
_RPT_A = _ROWS128 // 16
_BLK_ROWS = 256
_BLK_WORDS = _BLK_ROWS * _N
_SHARE = _BLK_WORDS // 16


@functools.cache
def _deg_kernel():
    return functools.partial(
        pl.kernel,
        out_type=jax.ShapeDtypeStruct((2 * _N,), jnp.float32),
        mesh=plsc.VectorSubcoreMesh(core_axis_name="c", subcore_axis_name="s"),
        compiler_params=pltpu.CompilerParams(needs_layout_passes=False),
        scratch_types=[
            pltpu.VMEM((_RPT_DEG * 128,), jnp.int32),
            pltpu.VMEM((_RPT_DEG, 128), jnp.int32),
            pltpu.VMEM((128,), jnp.float32),
            pltpu.VMEM((_N,), jnp.float32),
            pltpu.VMEM_SHARED((_N,), jnp.float32),
        ],
    )(_deg_body)


def _deg_body(dst_hbm, out_hbm, dstf, dstb, ones_v, zv, acc_sh):
    c = lax.axis_index("c")
    s = lax.axis_index("s")
    shard = c * 16 + s
    nwords = _RPT_DEG * 128

    def _fill(i, carry):
        ones_v[pl.ds(i * 16, 16)] = jnp.full((16,), 1.0, jnp.float32)
        return carry

    lax.fori_loop(0, 8, _fill, 0)

    def _zero(i, carry):
        zv[pl.ds(i * 16, 16)] = jnp.zeros((16,), jnp.float32)
        return carry

    lax.fori_loop(0, _N // 16, _zero, 0)

    @pl.when(s == 0)
    def _():
        pltpu.sync_copy(zv, acc_sh)

    plsc.subcore_barrier()
    pltpu.sync_copy(dst_hbm.at[pl.ds(shard * nwords, nwords)], dstf)

    def _pack(j, carry):
        for t in range(8):
            dstb[j, pl.ds(t * 16, 16)] = dstf[pl.ds(j * 128 + t * 16, 16)]
        return carry

    lax.fori_loop(0, _RPT_DEG, _pack, 0)

    def _scat(j, carry):
        pltpu.sync_copy(ones_v, acc_sh.at[dstb.at[j]], add=True)
        return carry

    lax.fori_loop(0, _RPT_DEG, _scat, 0)
    plsc.subcore_barrier()

    @pl.when(s == 0)
    def _():
        pltpu.sync_copy(acc_sh, out_hbm.at[pl.ds(c * _N, _N)])


@functools.cache
def _abuild_kernel():
    return functools.partial(
        pl.kernel,
        out_type=jax.ShapeDtypeStruct((_N * _N,), jnp.float32),
        mesh=plsc.VectorSubcoreMesh(core_axis_name="c", subcore_axis_name="s"),
        compiler_params=pltpu.CompilerParams(needs_layout_passes=False),
        scratch_types=[
            pltpu.VMEM((_N,), jnp.float32),
            pltpu.VMEM((_RPT_A * 128,), jnp.int32),
            pltpu.VMEM((_RPT_A * 128,), jnp.int32),
            pltpu.VMEM((_RPT_A * 128,), jnp.int32),
            pltpu.VMEM((_RPT_A * 128,), jnp.float32),
            pltpu.VMEM((_RPT_A, 128), jnp.int32),
            pltpu.VMEM((_RPT_A * 128,), jnp.float32),
            pltpu.VMEM((8192,), jnp.float32),
            pltpu.VMEM_SHARED((_BLK_WORDS,), jnp.float32),
            pltpu.SemaphoreType.DMA,
        ],
    )(_abuild_body)


def _abuild_body(src_hbm, dst_hbm, dis_hbm, a_hbm,
                 dis_v, srcb, dstb, gab, nrmb, idxb, valb, zv, acc_sh, sem):
    c = lax.axis_index("c")
    s = lax.axis_index("s")
    pltpu.sync_copy(dis_hbm, dis_v)

    def _zfill(i, carry):
        zv[pl.ds(i * 16, 16)] = jnp.zeros((16,), jnp.float32)
        return carry

    lax.fori_loop(0, 8192 // 16, _zfill, 0)

    nwords = _RPT_A * 128
    pltpu.sync_copy(src_hbm.at[pl.ds(s * nwords, nwords)], srcb)
    pltpu.sync_copy(dst_hbm.at[pl.ds(s * nwords, nwords)], dstb)

    def _pre(v, carry):
        off = v * 16
        sv = srcb[pl.ds(off, 16)]
        dv = dstb[pl.ds(off, 16)]
        nrm = plsc.load_gather(dis_v, [sv]) * plsc.load_gather(dis_v, [dv])
        ga = (dv << 12) + sv
        nrmb[pl.ds(off, 16)] = nrm
        gab[pl.ds(off, 16)] = ga
        return carry

    lax.fori_loop(0, nwords // 16, _pre, 0)

    def _pass(p, carry):
        boff = (c * 8 + p) * _BLK_WORDS

        zero_handles = [
            pltpu.async_copy(
                zv, acc_sh.at[pl.ds(s * _SHARE + q * 8192, 8192)], sem)
            for q in range(_SHARE // 8192)]

        def _cmp(j, c2):
            for t in range(8):
                off = j * 128 + t * 16
                ga = gab[pl.ds(off, 16)]
                idx = ga - boff
                inb = (idx >= 0) & (idx < _BLK_WORDS)
                idxs = jnp.where(inb, idx, ga & 1023)
                val = jnp.where(inb, nrmb[pl.ds(off, 16)],
                                jnp.zeros((16,), jnp.float32))
                idxb[j, pl.ds(t * 16, 16)] = idxs
                valb[pl.ds(off, 16)] = val
            return c2

        lax.fori_loop(0, _RPT_A, _cmp, 0)
        for h in zero_handles:
            h.wait()
        plsc.subcore_barrier()

        handles = []
        for j in range(_RPT_A):
            handles.append(pltpu.async_copy(
                valb.at[pl.ds(j * 128, 128)],
                acc_sh.at[idxb.at[j]], sem, add=True))
            if j >= 4:
                handles[j - 4].wait()
        for h in handles[-4:]:
            h.wait()
        plsc.subcore_barrier()
        pltpu.sync_copy(acc_sh.at[pl.ds(s * _SHARE, _SHARE)],
                        a_hbm.at[pl.ds(boff + s * _SHARE, _SHARE)])
        plsc.subcore_barrier()
        return carry

    lax.fori_loop(0, 8, _pass, 0)


def _dis_call(deg_parts):
    def body(deg_ref, o_ref):
        o_ref[...] = lax.rsqrt(jnp.sum(deg_ref[...], axis=0, keepdims=True))

    return pl.pallas_call(
        body, out_shape=jax.ShapeDtypeStruct((1, _N), jnp.float32)
    )(deg_parts)


def _matmul(a, b, *, bias=None, add_a=None, add_b=None, trans_b=False,
            relu=False, bm=1024, bn=512, bk=512, a_flat=False):
    if a_flat:
        K = b.shape[1] if trans_b else b.shape[0]
        M = a.size // K
    else:
        M, K = a.shape
    Nn = b.shape[0] if trans_b else b.shape[1]
    bm, bn, bk = min(bm, M), min(bn, Nn), min(bk, K)
    assert not a_flat or bk == K
    nk = K // bk
    grid = (M // bm, Nn // bn, nk)
    dims = (((1,), (1,)), ((), ())) if trans_b else (((1,), (0,)), ((), ()))

    if a_flat:
        a_spec = pl.BlockSpec((bm * K // 128, 128), lambda i, j, k: (i, 0))
    else:
        a_spec = pl.BlockSpec((bm, bk), lambda i, j, k: (i, k))
    b_spec = (pl.BlockSpec((bn, bk), lambda i, j, k: (j, k)) if trans_b
              else pl.BlockSpec((bk, bn), lambda i, j, k: (k, j)))
    in_specs = [a_spec, b_spec]
    args = [a, b]
    if add_a is not None:
        in_specs.append(a_spec)
        args.append(add_a)
    if add_b is not None:
        in_specs.append(b_spec)
        args.append(add_b)
    if bias is not None:
        in_specs.append(pl.BlockSpec((1, bn), lambda i, j, k: (0, j)))
        args.append(bias.reshape(1, Nn))

    has_add_a, has_add_b, has_bias = (add_a is not None, add_b is not None,
                                      bias is not None)

    def body(*refs):
        if nk > 1:
            *ins, o_ref, acc_ref = refs
        else:
            *ins, o_ref = refs
            acc_ref = None
        it = iter(ins)
        a_ref, b_ref = next(it), next(it)
        a2_ref = next(it) if has_add_a else None
        b2_ref = next(it) if has_add_b else None
        bias_ref = next(it) if has_bias else None

        av = a_ref.reshape(bm, bk)[...] if a_flat else a_ref[...]
        if has_add_a:
            av = av + a2_ref[...]
        bv = b_ref[...]
        if has_add_b:
            bv = bv + b2_ref[...]
        part = lax.dot_general(av, bv, dims,
                               preferred_element_type=jnp.float32)

        def _epilogue(r):
            if has_bias:
                r = r + bias_ref[...]
            if relu:
                r = jnp.maximum(r, 0.0)
            o_ref[...] = r

        if nk > 1:
            k = pl.program_id(2)

            @pl.when(k == 0)
            def _():
                acc_ref[...] = part

            @pl.when(k > 0)
            def _():
                acc_ref[...] += part

            @pl.when(k == nk - 1)
            def _():
                _epilogue(acc_ref[...])
        else:
            _epilogue(part)

    scratch = [pltpu.VMEM((bm, bn), jnp.float32)] if nk > 1 else []
    return pl.pallas_call(
        body, grid=grid,
        in_specs=in_specs,
        out_specs=pl.BlockSpec((bm, bn), lambda i, j, k: (i, j)),
        out_shape=jax.ShapeDtypeStruct((M, Nn), jnp.float32),
        scratch_shapes=scratch,
        compiler_params=pltpu.CompilerParams(
            dimension_semantics=("parallel", "parallel", "arbitrary"),
            vmem_limit_bytes=100 * 1024 * 1024),
    )(*args)


def _mlp_chain(B, ws, bs, bm=512):
    grid = (_N // bm,)
    in_specs = [pl.BlockSpec((bm, _N), lambda i: (i, 0))]
    args = [B]
    for w, b in zip(ws, bs):
        in_specs.append(pl.BlockSpec(w.shape, lambda i: (0, 0)))
        args.append(w)
        in_specs.append(pl.BlockSpec((1, b.shape[0]), lambda i: (0, 0)))
        args.append(b.reshape(1, -1))

    out_shapes = [jax.ShapeDtypeStruct((_N, ws[0].shape[1]), jnp.float32),
                  jax.ShapeDtypeStruct((_N, ws[1].shape[1]), jnp.float32),
                  jax.ShapeDtypeStruct((_N, ws[2].shape[1]), jnp.float32),
                  jax.ShapeDtypeStruct((_N, ws[5].shape[1]), jnp.float32)]
    out_specs = [pl.BlockSpec((bm, s.shape[1]), lambda i: (i, 0))
                 for s in out_shapes]

    def body(*refs):
        b_ref = refs[0]
        wrefs = refs[1:13]
        h1_ref, h2_ref, h3_ref, bhat_ref = refs[13:]
        h = b_ref[...]
        outs = []
        for li in range(6):
            h = jnp.maximum(
                jnp.dot(h, wrefs[2 * li][...],
                        preferred_element_type=jnp.float32)
                + wrefs[2 * li + 1][...], 0.0)
            outs.append(h)
        h1_ref[...] = outs[0]
        h2_ref[...] = outs[1]
        h3_ref[...] = outs[2]
        bhat_ref[...] = outs[5]

    return pl.pallas_call(
        body, grid=grid,
        in_specs=in_specs,
        out_specs=out_specs,
        out_shape=out_shapes,
        compiler_params=pltpu.CompilerParams(
            dimension_semantics=("parallel",),
            vmem_limit_bytes=100 * 1024 * 1024),
    )(*args)


def _a_stage(A, rhs, *, w1=None, bias=None, relu=False, add=None, w2=None,
             extra_w=None, bm=512):
    Kr = rhs.shape[1]
    M = _N
    n_mid = w1.shape[1] if w1 is not None else Kr
    n_out = w2.shape[1] if w2 is not None else n_mid
    grid = (M // bm,)

    in_specs = [pl.BlockSpec((bm * _N // 128, 128), lambda i: (i, 0)),
                pl.BlockSpec((_N, Kr), lambda i: (0, 0))]
    args = [A, rhs]
    if w1 is not None:
        in_specs.append(pl.BlockSpec(w1.shape, lambda i: (0, 0)))
        args.append(w1)
    if bias is not None:
        in_specs.append(pl.BlockSpec((1, n_mid), lambda i: (0, 0)))
        args.append(bias.reshape(1, n_mid))
    if add is not None:
        in_specs.append(pl.BlockSpec((bm, n_mid), lambda i: (i, 0)))
        args.append(add)
    if w2 is not None:
        in_specs.append(pl.BlockSpec(w2.shape, lambda i: (0, 0)))
        args.append(w2)
    if extra_w is not None:
        in_specs.append(pl.BlockSpec(extra_w.shape, lambda i: (0, 0)))
        args.append(extra_w)

    out_shapes = [jax.ShapeDtypeStruct((M, n_out), jnp.float32)]
    out_specs = [pl.BlockSpec((bm, n_out), lambda i: (i, 0))]
    if extra_w is not None:
        out_shapes.append(
            jax.ShapeDtypeStruct((M, extra_w.shape[1]), jnp.float32))
        out_specs.append(
            pl.BlockSpec((bm, extra_w.shape[1]), lambda i: (i, 0)))

    has = (w1 is not None, bias is not None, add is not None,
           w2 is not None, extra_w is not None)

    def body(*refs):
        nouts = 2 if extra_w is not None else 1
        ins, outs = refs[:-nouts], refs[-nouts:]
        it = iter(ins)
        a_ref, rhs_ref = next(it), next(it)
        w1_ref = next(it) if has[0] else None
        b_ref = next(it) if has[1] else None
        add_ref = next(it) if has[2] else None
        w2_ref = next(it) if has[3] else None
        ew_ref = next(it) if has[4] else None

        acc = lax.dot_general(a_ref.reshape(bm, _N)[...], rhs_ref[...],
                              (((1,), (0,)), ((), ())),
                              preferred_element_type=jnp.float32)
        if has[0]:
            acc = jnp.dot(acc, w1_ref[...],
                          preferred_element_type=jnp.float32)
        if has[1]:
            acc = acc + b_ref[...]
        if relu:
            acc = jnp.maximum(acc, 0.0)
        if has[4]:
            outs[1][...] = jnp.dot(acc, ew_ref[...],
                                   preferred_element_type=jnp.float32)
        if has[2]:
            acc = acc + add_ref[...]
        if has[3]:
            acc = jnp.dot(acc, w2_ref[...],
                          preferred_element_type=jnp.float32)
        outs[0][...] = acc

    res = pl.pallas_call(
        body, grid=grid,
        in_specs=in_specs,
        out_specs=out_specs if extra_w is not None else out_specs[0],
        out_shape=out_shapes if extra_w is not None else out_shapes[0],
        compiler_params=pltpu.CompilerParams(
            dimension_semantics=("parallel",),
            vmem_limit_bytes=100 * 1024 * 1024),
    )(*args)
    return res


def kernel(x, edge_index, B, W_enc0, b_enc0, W_enc1, b_enc1, W_enc2, b_enc2,
           W_dec0, b_dec0, W_dec1, b_dec1, W_dec2, b_dec2,
           W_gcn0, b_gcn0, W_gcn1, b_gcn1, W_gcn2, b_gcn2, W_gcn3, b_gcn3,
           W_attr, b_attr):
    loop = jnp.arange(_N, dtype=edge_index.dtype)
    src_e = jnp.concatenate([edge_index[0], loop])
    dst_e = jnp.concatenate([edge_index[1], loop])

    deg_parts = _deg_kernel()(dst_e).reshape(2, _N)
    dis = _dis_call(deg_parts).reshape(_N)
    A = _abuild_kernel()(src_e, dst_e, dis).reshape(_N * 32, 128)

    h1, h2, h3, B_hat = _mlp_chain(
        B,
        (W_enc0, W_enc1, W_enc2, W_dec0, W_dec1, W_dec2),
        (b_enc0, b_enc1, b_enc2, b_dec0, b_dec1, b_dec2))

    t1 = _a_stage(A, x, w1=W_gcn0, bias=b_gcn0, relu=True, add=h1)
    u2 = _a_stage(A, t1, w1=W_gcn1, bias=b_gcn1, relu=True, add=h2, w2=W_gcn2)
    u3 = _a_stage(A, u2, bias=b_gcn2, relu=True, add=h3, w2=W_gcn3)
    z, ua = _a_stage(A, u3, bias=b_gcn3, relu=True, extra_w=W_attr)
    X_hat = _a_stage(A, ua, bias=b_attr)

    A_hat = _matmul(z, z, trans_b=True, bm=1024, bn=1024)

    return (A_hat, B_hat, X_hat, h3, z)
